# Initial kernel scaffold; baseline (speedup 1.0000x reference)
import jax
import jax.numpy as jnp
from jax.experimental import pallas as pl


def kernel(class_indices, camera_pos):
    raise NotImplementedError("write your pallas kernel here")


# speedup vs baseline: 2.1139x; 2.1139x over previous
import functools

import jax
import jax.numpy as jnp
from jax import lax
from jax.experimental import pallas as pl
from jax.experimental.pallas import tpu as pltpu
from jax.experimental.pallas import tpu_sc as plsc

NUM_CLASSES_ = 100000
NUM_VIEWS_ = 8
BATCH_ = 16384
ROW = NUM_VIEWS_ * 3

_info = plsc.get_sparse_core_info()
NC, NS = _info.num_cores, _info.num_subcores
NW = NC * NS
B_PER_W = BATCH_ // NW
CHUNK = 128
NCHUNK = B_PER_W // CHUNK


@functools.partial(
    pl.kernel,
    mesh=plsc.VectorSubcoreMesh(core_axis_name="c", subcore_axis_name="s"),
    out_type=jax.ShapeDtypeStruct((NW, NCHUNK, CHUNK, ROW), jnp.float32),
    scratch_types=[
        pltpu.VMEM((NCHUNK, CHUNK), jnp.int32),
        pltpu.VMEM((NCHUNK, CHUNK, ROW), jnp.float32),
        pltpu.SemaphoreType.DMA,
    ],
    compiler_params=pltpu.CompilerParams(use_tc_tiling_on_sc=False),
)
def _gather_sc(idx_hbm, table_hbm, out_hbm, idx_v, rows_v, sem):
    wid = lax.axis_index("s") * NC + lax.axis_index("c")
    pltpu.sync_copy(idx_hbm.at[wid], idx_v)
    copies = []
    for j in range(NCHUNK):
        copies.append(
            pltpu.async_copy(table_hbm.at[idx_v.at[j]], rows_v.at[j], sem)
        )
    for c in copies:
        c.wait()
    pltpu.sync_copy(rows_v, out_hbm.at[wid])


def kernel(class_indices, camera_pos):
    idx = class_indices.astype(jnp.int32).reshape(NW, NCHUNK, CHUNK)
    table = camera_pos.reshape(NUM_CLASSES_, ROW)
    out = _gather_sc(idx, table)
    return out.reshape(BATCH_, NUM_VIEWS_, 3)


# speedup vs baseline: 5.8705x; 2.7771x over previous
import functools

import jax
import jax.numpy as jnp
from jax import lax
from jax.experimental import pallas as pl
from jax.experimental.pallas import tpu as pltpu
from jax.experimental.pallas import tpu_sc as plsc

NUM_CLASSES_ = 100000
NUM_VIEWS_ = 8
BATCH_ = 16384
NPLANE = NUM_VIEWS_ * 3
HALF = BATCH_ // 2

_info = plsc.get_sparse_core_info()
NC, NS = _info.num_cores, _info.num_subcores


@functools.partial(
    pl.kernel,
    mesh=plsc.VectorSubcoreMesh(core_axis_name="c", subcore_axis_name="s"),
    out_type=jax.ShapeDtypeStruct((3, NUM_VIEWS_, BATCH_), jnp.float32),
    scratch_types=[
        pltpu.VMEM((NUM_CLASSES_,), jnp.float32),
        pltpu.VMEM((HALF,), jnp.int32),
        pltpu.VMEM((HALF,), jnp.float32),
    ],
    compiler_params=pltpu.CompilerParams(
        use_tc_tiling_on_sc=False, needs_layout_passes=False
    ),
)
def _gather_planes(idx_hbm, table_hbm, out_hbm, plane_v, idx_v, out_v):
    wid = lax.axis_index("s") * NC + lax.axis_index("c")

    @pl.when(wid < NPLANE)
    def _():
        c = wid // NUM_VIEWS_
        v = wid % NUM_VIEWS_
        pltpu.sync_copy(table_hbm.at[c, v], plane_v)
        for h in range(2):
            pltpu.sync_copy(idx_hbm.at[pl.ds(h * HALF, HALF)], idx_v)

            def body(k, carry):
                ii = idx_v[pl.ds(k * 16, 16)]
                out_v[pl.ds(k * 16, 16)] = plsc.load_gather(plane_v, [ii])
                return carry

            lax.fori_loop(0, HALF // 16, body, 0, unroll=4)
            pltpu.sync_copy(out_v, out_hbm.at[c, v, pl.ds(h * HALF, HALF)])


def kernel(class_indices, camera_pos):
    idx = class_indices.astype(jnp.int32)
    tab = camera_pos.transpose(2, 1, 0)
    out = _gather_planes(idx, tab)
    return out.transpose(2, 1, 0)


# speedup vs baseline: 8.0632x; 1.3735x over previous
import functools

import jax
import jax.numpy as jnp
from jax import lax
from jax.experimental import pallas as pl
from jax.experimental.pallas import tpu as pltpu
from jax.experimental.pallas import tpu_sc as plsc

NUM_CLASSES_ = 100000
NUM_VIEWS_ = 8
BATCH_ = 16384
NPLANE = NUM_VIEWS_ * 3
HALF = BATCH_ // 2

_info = plsc.get_sparse_core_info()
NC, NS = _info.num_cores, _info.num_subcores


@functools.partial(
    pl.kernel,
    mesh=plsc.VectorSubcoreMesh(core_axis_name="c", subcore_axis_name="s"),
    out_type=jax.ShapeDtypeStruct((3, NUM_VIEWS_, BATCH_), jnp.float32),
    scratch_types=[
        pltpu.VMEM((NUM_CLASSES_,), jnp.float32),
        pltpu.VMEM((HALF,), jnp.int32),
        pltpu.VMEM((HALF,), jnp.float32),
    ],
    compiler_params=pltpu.CompilerParams(
        use_tc_tiling_on_sc=True, needs_layout_passes=False
    ),
)
def _gather_planes(idx_hbm, table_hbm, out_hbm, plane_v, idx_v, out_v):
    wid = lax.axis_index("s") * NC + lax.axis_index("c")

    @pl.when(wid < NPLANE)
    def _():
        c = wid // NUM_VIEWS_
        v = wid % NUM_VIEWS_
        pltpu.sync_copy(table_hbm.at[c, v], plane_v)
        for h in range(2):
            pltpu.sync_copy(idx_hbm.at[pl.ds(h * HALF, HALF)], idx_v)

            def body(k, carry):
                ii = idx_v[pl.ds(k * 16, 16)]
                out_v[pl.ds(k * 16, 16)] = plsc.load_gather(plane_v, [ii])
                return carry

            lax.fori_loop(0, HALF // 16, body, 0, unroll=4)
            pltpu.sync_copy(out_v, out_hbm.at[c, v, pl.ds(h * HALF, HALF)])


def kernel(class_indices, camera_pos):
    idx = class_indices.astype(jnp.int32)
    tab = camera_pos.transpose(2, 1, 0)
    out = _gather_planes(idx, tab)
    return out.transpose(2, 1, 0)
